# baseline (speedup 1.0000x reference)
import functools

import jax
import jax.numpy as jnp
from jax import lax
from jax.experimental import pallas as pl
from jax.experimental.pallas import tpu as pltpu
from jax.experimental.pallas import tpu_sc as plsc

_N = 10000
_K = 32
_D = 128
_NW = 32
_NSUB = 16
_N_PAD = 10240
_N_CALLS = 4
_N_CALL = _N_PAD // _N_CALLS
_NODES_PER_W = _N_CALL // _NW
_EMB_PAD = 10240
_STAGE_ROWS = _EMB_PAD // _NSUB
_NODES_PER_BATCH = 4
_ROWS_PER_BATCH = _NODES_PER_BATCH * _K
_N_BATCH = _NODES_PER_W // _NODES_PER_BATCH
_NBUF = 2
_N_BATCH_PAD = _N_BATCH + 2 * _NBUF
_ACC_ROWS = (_N_BATCH + _NBUF) * _NODES_PER_BATCH
_LANES = 16


def _tree_reduce_batch(buf, acc_v, base):
    for j in range(_NODES_PER_BATCH):
        for c in range(_D // _LANES):
            vals = [
                buf[j * _K + r, pl.ds(c * _LANES, _LANES)] for r in range(_K)
            ]
            while len(vals) > 1:
                nxt = [vals[i] + vals[i + 1] for i in range(0, len(vals) - 1, 2)]
                if len(vals) % 2:
                    nxt.append(vals[-1])
                vals = nxt
            acc_v[base + j, pl.ds(c * _LANES, _LANES)] = vals[0]


def _sc_gather_sum(idx3, emb):
    mesh = plsc.VectorSubcoreMesh(core_axis_name="c", subcore_axis_name="s")

    @functools.partial(
        pl.kernel,
        mesh=mesh,
        out_type=jax.ShapeDtypeStruct((_NW, _NODES_PER_W, _D), jnp.float32),
        scratch_types=[
            pltpu.VMEM_SHARED((_EMB_PAD, _D), jnp.float32),
            pltpu.VMEM((_N_BATCH_PAD, _ROWS_PER_BATCH), jnp.int32),
            pltpu.VMEM((_ROWS_PER_BATCH, _D), jnp.float32),
            pltpu.VMEM((_ROWS_PER_BATCH, _D), jnp.float32),
            pltpu.VMEM((_ACC_ROWS, _D), jnp.float32),
            pltpu.SemaphoreType.DMA,
            pltpu.SemaphoreType.DMA,
        ],
    )
    def k(idx_hbm, emb_hbm, out_hbm, emb_s, idx_v, b0, b1, acc_v, s0, s1):
        bufs = (b0, b1)
        sems = (s0, s1)
        sid = lax.axis_index("s")
        wid = sid * 2 + lax.axis_index("c")

        row0 = sid * _STAGE_ROWS
        for t in range(_STAGE_ROWS // _ROWS_PER_BATCH):
            buf = bufs[t % _NBUF]
            pltpu.sync_copy(
                emb_hbm.at[pl.ds(row0 + t * _ROWS_PER_BATCH, _ROWS_PER_BATCH)],
                buf,
            )
            pltpu.sync_copy(
                buf,
                emb_s.at[pl.ds(row0 + t * _ROWS_PER_BATCH, _ROWS_PER_BATCH)],
            )
        pltpu.sync_copy(idx_hbm.at[wid], idx_v)
        plsc.subcore_barrier()

        def start(gb, slot):
            pltpu.async_copy(emb_s.at[idx_v.at[gb]], bufs[slot], sems[slot])

        def wait(gb, slot):
            pltpu.make_async_copy(
                emb_s.at[idx_v.at[gb]], bufs[slot], sems[slot]
            ).wait()

        for b in range(_NBUF):
            start(b, b)

        @pl.loop(0, _N_BATCH + _NBUF, step=_NBUF)
        def _(g):
            for b in range(_NBUF):
                wait(g + b, b)
                _tree_reduce_batch(bufs[b], acc_v, (g + b) * _NODES_PER_BATCH)
                start(g + b + _NBUF, b)

        for b in range(_NBUF):
            wait(_N_BATCH + _NBUF + b, b)

        pltpu.sync_copy(acc_v.at[pl.ds(0, _NODES_PER_W)], out_hbm.at[wid])

    return k(idx3, emb)


def _tc_linear_norm(sums, wt):
    blk = min(1024, _N_CALL)

    def body(x_ref, w_ref, o_ref):
        x = x_ref[...] * (1.0 / _K)
        y = jnp.dot(x, w_ref[...], preferred_element_type=jnp.float32)
        y = jnp.maximum(y, 0.0)
        n = jnp.sqrt(jnp.sum(y * y, axis=1, keepdims=True))
        o_ref[...] = y / jnp.maximum(n, 1e-12)

    return pl.pallas_call(
        body,
        grid=(max(_N_CALL // blk, 1),),
        in_specs=[
            pl.BlockSpec((blk, _D), lambda i: (i, 0)),
            pl.BlockSpec((_D, _D), lambda i: (0, 0)),
        ],
        out_specs=pl.BlockSpec((blk, _D), lambda i: (i, 0)),
        out_shape=jax.ShapeDtypeStruct((_N_CALL, _D), jnp.float32),
    )(sums, wt)


def kernel(neighbors, emb_features, W):
    nb = neighbors.astype(jnp.int32).reshape(-1)
    nb = jnp.concatenate([nb, jnp.zeros((_N_PAD * _K - _N * _K,), jnp.int32)])
    idx5 = nb.reshape(_N_CALLS, _NW, _N_BATCH, _ROWS_PER_BATCH)
    pad = jnp.zeros(
        (_NW, _N_BATCH_PAD - _N_BATCH, _ROWS_PER_BATCH), jnp.int32
    )
    emb_p = jnp.concatenate(
        [emb_features, jnp.zeros((_EMB_PAD - _N, _D), jnp.float32)]
    )
    wt = W.T
    parts = []
    for cc in range(_N_CALLS):
        idx3 = jnp.concatenate([idx5[cc], pad], axis=1)
        sums = _sc_gather_sum(idx3, emb_p).reshape(_N_CALL, _D)
        parts.append(_tc_linear_norm(sums, wt))
    return jnp.concatenate(parts, axis=0)[:_N]


# speedup vs baseline: 1.4453x; 1.4453x over previous
import functools

import jax
import jax.numpy as jnp
from jax import lax
from jax.experimental import pallas as pl
from jax.experimental.pallas import tpu as pltpu
from jax.experimental.pallas import tpu_sc as plsc

_N = 10000
_K = 32
_D = 128
_NW = 32
_NSUB = 16
_N_PAD = 10240
_N_CALLS = 2
_N_CALL = _N_PAD // _N_CALLS
_NODES_PER_W = _N_CALL // _NW
_EMB_PAD = 10240
_STAGE_ROWS = _EMB_PAD // _NSUB
_NODES_PER_BATCH = 4
_ROWS_PER_BATCH = _NODES_PER_BATCH * _K
_N_BATCH = _NODES_PER_W // _NODES_PER_BATCH
_LANES = 16


def _tree_reduce_batch(buf, acc_v, base):
    for j in range(_NODES_PER_BATCH):
        for c in range(_D // _LANES):
            vals = [
                buf[j * _K + r, pl.ds(c * _LANES, _LANES)] for r in range(_K)
            ]
            while len(vals) > 1:
                nxt = [vals[i] + vals[i + 1] for i in range(0, len(vals) - 1, 2)]
                if len(vals) % 2:
                    nxt.append(vals[-1])
                vals = nxt
            acc_v[base + j, pl.ds(c * _LANES, _LANES)] = vals[0]


def _sc_gather_sum(idx3, emb):
    mesh = plsc.VectorSubcoreMesh(core_axis_name="c", subcore_axis_name="s")

    @functools.partial(
        pl.kernel,
        mesh=mesh,
        out_type=jax.ShapeDtypeStruct((_NW, _NODES_PER_W, _D), jnp.float32),
        scratch_types=[
            pltpu.VMEM_SHARED((_EMB_PAD, _D), jnp.float32),
            pltpu.VMEM((_N_BATCH, _ROWS_PER_BATCH), jnp.int32),
            pltpu.VMEM((_ROWS_PER_BATCH, _D), jnp.float32),
            pltpu.VMEM((_NODES_PER_W, _D), jnp.float32),
            pltpu.SemaphoreType.DMA,
        ],
    )
    def k(idx_hbm, emb_hbm, out_hbm, emb_s, idx_v, buf_v, acc_v, sem):
        sid = lax.axis_index("s")
        wid = sid * 2 + lax.axis_index("c")

        row0 = sid * _STAGE_ROWS
        for t in range(_STAGE_ROWS // _ROWS_PER_BATCH):
            pltpu.sync_copy(
                emb_hbm.at[pl.ds(row0 + t * _ROWS_PER_BATCH, _ROWS_PER_BATCH)],
                buf_v,
            )
            pltpu.sync_copy(
                buf_v,
                emb_s.at[pl.ds(row0 + t * _ROWS_PER_BATCH, _ROWS_PER_BATCH)],
            )
        pltpu.sync_copy(idx_hbm.at[wid], idx_v)
        plsc.subcore_barrier()

        @pl.loop(0, _N_BATCH)
        def _(g):
            pltpu.async_copy(emb_s.at[idx_v.at[g]], buf_v, sem).wait()
            _tree_reduce_batch(buf_v, acc_v, g * _NODES_PER_BATCH)

        pltpu.sync_copy(acc_v, out_hbm.at[wid])

    return k(idx3, emb)


def _tc_linear_norm(sums, wt):
    blk = min(1024, _N_CALL)

    def body(x_ref, w_ref, o_ref):
        x = x_ref[...] * (1.0 / _K)
        y = jnp.dot(x, w_ref[...], preferred_element_type=jnp.float32)
        y = jnp.maximum(y, 0.0)
        n = jnp.sqrt(jnp.sum(y * y, axis=1, keepdims=True))
        o_ref[...] = y / jnp.maximum(n, 1e-12)

    return pl.pallas_call(
        body,
        grid=(max(_N_CALL // blk, 1),),
        in_specs=[
            pl.BlockSpec((blk, _D), lambda i: (i, 0)),
            pl.BlockSpec((_D, _D), lambda i: (0, 0)),
        ],
        out_specs=pl.BlockSpec((blk, _D), lambda i: (i, 0)),
        out_shape=jax.ShapeDtypeStruct((_N_CALL, _D), jnp.float32),
    )(sums, wt)


def kernel(neighbors, emb_features, W):
    nb = neighbors.astype(jnp.int32).reshape(-1)
    nb = jnp.concatenate([nb, jnp.zeros((_N_PAD * _K - _N * _K,), jnp.int32)])
    idx5 = nb.reshape(_N_CALLS, _NW, _N_BATCH, _ROWS_PER_BATCH)
    emb_p = jnp.concatenate(
        [emb_features, jnp.zeros((_EMB_PAD - _N, _D), jnp.float32)]
    )
    wt = W.T
    parts = []
    for cc in range(_N_CALLS):
        sums = _sc_gather_sum(idx5[cc], emb_p).reshape(_N_CALL, _D)
        parts.append(_tc_linear_norm(sums, wt))
    return jnp.concatenate(parts, axis=0)[:_N]
